# Initial kernel scaffold; baseline (speedup 1.0000x reference)
import jax
import jax.numpy as jnp
from jax.experimental import pallas as pl


def kernel(x, edge_index, edge_weight, W, b):
    raise NotImplementedError("write your pallas kernel here")


# speedup vs baseline: 3.8151x; 3.8151x over previous
import functools

import jax
import jax.numpy as jnp
from jax import lax
from jax.experimental import pallas as pl
from jax.experimental.pallas import tpu as pltpu
from jax.experimental.pallas import tpu_sc as plsc

NC = 2
NS = 16
L = 16
NW = NC * NS

K = 128
C = 79
RPT = 640
NP = NS * RPT
D = 128
NGRP = D // L


def _make_sc_kernel():
    mesh = plsc.VectorSubcoreMesh(core_axis_name="c", subcore_axis_name="s")

    @functools.partial(
        pl.kernel,
        mesh=mesh,
        out_type=jax.ShapeDtypeStruct((NC, NP, D), jnp.float32),
        scratch_types=[
            pltpu.VMEM((C, K), jnp.float32),
            pltpu.VMEM((K, D), jnp.float32),
            pltpu.VMEM((K,), jnp.int32),
            pltpu.VMEM((K,), jnp.int32),
            pltpu.VMEM_SHARED((NP, D), jnp.float32),
            pltpu.SemaphoreType.DMA,
        ],
    )
    def sc_kernel(x_hbm, src_hbm, dst_hbm, w_hbm, out_hbm,
                  w_v, rows_v, sidx_v, didx_v, acc, sem):
        cid = lax.axis_index("c")
        sid = lax.axis_index("s")
        wid = cid * NS + sid

        pltpu.sync_copy(w_hbm.at[wid], w_v)

        def _zero_row(r, _):
            for g in range(NGRP):
                rows_v[r, pl.ds(g * L, L)] = jnp.zeros((L,), jnp.float32)
            return 0

        lax.fori_loop(0, K, _zero_row, 0)
        base = sid * RPT
        for t in range(RPT // K):
            pltpu.sync_copy(rows_v, acc.at[pl.ds(base + t * K, K)])
        plsc.subcore_barrier()

        def _chunk(c, _):
            pltpu.sync_copy(src_hbm.at[wid, c], sidx_v)
            pltpu.sync_copy(dst_hbm.at[wid, c], didx_v)
            pltpu.async_copy(x_hbm.at[sidx_v], rows_v, sem).wait()

            def _scale_block(bi, _):
                wvec = w_v[c, pl.ds(bi * L, L)]
                for j in range(L):
                    wv = jnp.full((L,), wvec[j], jnp.float32)
                    e = bi * L + j
                    for g in range(NGRP):
                        sl = pl.ds(g * L, L)
                        rows_v[e, sl] = rows_v[e, sl] * wv
                return 0

            lax.fori_loop(0, K // L, _scale_block, 0)
            pltpu.sync_copy(rows_v, acc.at[didx_v], add=True)
            return 0

        lax.fori_loop(0, C, _chunk, 0)
        plsc.subcore_barrier()

        for t in range(RPT // K):
            pltpu.sync_copy(acc.at[pl.ds(base + t * K, K)], rows_v)
            pltpu.sync_copy(rows_v, out_hbm.at[cid, pl.ds(base + t * K, K)])

    return sc_kernel


def _tc_body(p_ref, w_ref, b_ref, o_ref):
    a = p_ref[0] + p_ref[1]
    y = lax.dot_general(a, w_ref[...], (((1,), (1,)), ((), ())),
                        preferred_element_type=jnp.float32,
                        precision=lax.Precision.HIGHEST)
    o_ref[...] = y + b_ref[...]


def kernel(x, edge_index, edge_weight, W, b):
    n = x.shape[0]
    e = edge_weight.shape[0]
    ep = NW * C * K
    src = jnp.pad(edge_index[1].astype(jnp.int32), (0, ep - e)).reshape(NW, C, K)
    dst = jnp.pad(edge_index[0].astype(jnp.int32), (0, ep - e)).reshape(NW, C, K)
    w = jnp.pad(edge_weight, (0, ep - e)).reshape(NW, C, K)

    partials = _make_sc_kernel()(x, src, dst, w)

    rblk = 400
    out = pl.pallas_call(
        _tc_body,
        grid=(n // rblk,),
        in_specs=[
            pl.BlockSpec((NC, rblk, D), lambda i: (0, i, 0)),
            pl.BlockSpec((D, D), lambda i: (0, 0)),
            pl.BlockSpec((1, D), lambda i: (0, 0)),
        ],
        out_specs=pl.BlockSpec((rblk, D), lambda i: (i, 0)),
        out_shape=jax.ShapeDtypeStruct((n, D), jnp.float32),
    )(partials, W, b.reshape(1, D))
    return out
